# baseline (speedup 1.0000x reference)
import jax
import jax.numpy as jnp
from jax.experimental import pallas as pl
from jax.experimental.pallas import tpu as pltpu

NFEAT = 128
NHID = 32
NCLASS = 16
NHEADS = 8
ALPHA = 0.2
B = 4
N = 1024


def _gat_kernel(x_ref, xt_ref, w_ref, c1_ref, c2_ref, fc_ref, out_ref):
    h_idx = pl.program_id(1)

    x = x_ref[0]
    xt = xt_ref[0]
    w = w_ref[0]
    hm = jnp.dot(x, w, preferred_element_type=jnp.float32)

    f1 = jnp.dot(x, c1_ref[0], preferred_element_type=jnp.float32,
                 precision=jax.lax.Precision.HIGHEST)
    f2r = jnp.dot(c2_ref[0], xt, preferred_element_type=jnp.float32,
                  precision=jax.lax.Precision.HIGHEST)

    mx = jnp.max(f2r)
    m = f1 + mx
    m = jnp.where(m > 0, m, ALPHA * m)
    g1 = f1 - m
    g1a = ALPHA * f1 - m
    g2a = ALPHA * f2r
    p = jnp.exp(jnp.maximum(g1 + f2r, g1a + g2a))
    hm_ext = jnp.concatenate([hm, jnp.ones((N, 1), jnp.float32)], axis=1)
    num = jnp.dot(p, hm_ext, preferred_element_type=jnp.float32)
    hp = num[:, :NHID] / num[:, NHID:]
    hp = jnp.where(hp > 0, hp, jnp.exp(jnp.minimum(hp, 0.0)) - 1.0)
    pooled = jnp.max(hp, axis=0, keepdims=True)

    contrib = jnp.dot(pooled, fc_ref[0], preferred_element_type=jnp.float32)

    @pl.when(h_idx == 0)
    def _():
        out_ref[0] = contrib

    @pl.when(h_idx != 0)
    def _():
        out_ref[0] += contrib


def kernel(x, W, a, FC):
    hp_ = jax.lax.Precision.HIGHEST
    c1 = jnp.einsum('hfo,ho->hf', W, a[:, :NHID, 0], precision=hp_)[:, :, None]
    c2 = jnp.einsum('hfo,ho->hf', W, a[:, NHID:, 0], precision=hp_)[:, None, :]
    xt = jnp.swapaxes(x, 1, 2)
    fc3d = FC.reshape(NHEADS, NHID, NCLASS)

    out = pl.pallas_call(
        _gat_kernel,
        grid=(B, NHEADS),
        in_specs=[
            pl.BlockSpec((1, N, NFEAT), lambda b, h: (b, 0, 0)),
            pl.BlockSpec((1, NFEAT, N), lambda b, h: (b, 0, 0)),
            pl.BlockSpec((1, NFEAT, NHID), lambda b, h: (h, 0, 0)),
            pl.BlockSpec((1, NFEAT, 1), lambda b, h: (h, 0, 0)),
            pl.BlockSpec((1, 1, NFEAT), lambda b, h: (h, 0, 0)),
            pl.BlockSpec((1, NHID, NCLASS), lambda b, h: (h, 0, 0)),
        ],
        out_specs=pl.BlockSpec((1, 1, NCLASS), lambda b, h: (b, 0, 0)),
        out_shape=jax.ShapeDtypeStruct((B, 1, NCLASS), jnp.float32),
        compiler_params=pltpu.CompilerParams(
            dimension_semantics=("parallel", "arbitrary"),
        ),
    )(x, xt, W, c1, c2, fc3d)
    return out.reshape(B, NCLASS)


# speedup vs baseline: 1.2598x; 1.2598x over previous
import jax
import jax.numpy as jnp
from jax.experimental import pallas as pl
from jax.experimental.pallas import tpu as pltpu

NFEAT = 128
NHID = 32
NCLASS = 16
NHEADS = 8
ALPHA = 0.2
B = 4
N = 1024


def _gat_kernel(x_ref, xt_ref, w_ref, wt_ref, a1_ref, a2_ref, fc_ref, out_ref):
    h_idx = pl.program_id(1)

    x = x_ref[0]
    xt = xt_ref[0]
    w = w_ref[0]
    wt = wt_ref[0]
    hm = jnp.dot(x, w, preferred_element_type=jnp.float32)
    hmt = jnp.dot(wt, xt, preferred_element_type=jnp.float32)

    f1 = jnp.dot(hm, a1_ref[0], preferred_element_type=jnp.float32)
    f2r = jnp.dot(a2_ref[0], hmt, preferred_element_type=jnp.float32)

    mx = jnp.max(f2r)
    m = f1 + mx
    m = jnp.where(m > 0, m, ALPHA * m)
    g1 = f1 - m
    g1a = ALPHA * f1 - m
    g2a = ALPHA * f2r
    p = jnp.exp(jnp.maximum(g1 + f2r, g1a + g2a))
    hm_ext = jnp.concatenate([hm, jnp.ones((N, 1), jnp.float32)], axis=1)
    num = jnp.dot(p, hm_ext, preferred_element_type=jnp.float32)
    hp = num[:, :NHID] / num[:, NHID:]
    hp = jnp.where(hp > 0, hp, jnp.exp(jnp.minimum(hp, 0.0)) - 1.0)
    pooled = jnp.max(hp, axis=0, keepdims=True)

    contrib = jnp.dot(pooled, fc_ref[0], preferred_element_type=jnp.float32)

    @pl.when(h_idx == 0)
    def _():
        out_ref[0] = contrib

    @pl.when(h_idx != 0)
    def _():
        out_ref[0] += contrib


def kernel(x, W, a, FC):
    a1 = a[:, :NHID, :]
    a2 = jnp.swapaxes(a[:, NHID:, :], 1, 2)
    wt = jnp.swapaxes(W, 1, 2)
    xt = jnp.swapaxes(x, 1, 2)
    fc3d = FC.reshape(NHEADS, NHID, NCLASS)

    out = pl.pallas_call(
        _gat_kernel,
        grid=(B, NHEADS),
        in_specs=[
            pl.BlockSpec((1, N, NFEAT), lambda b, h: (b, 0, 0)),
            pl.BlockSpec((1, NFEAT, N), lambda b, h: (b, 0, 0)),
            pl.BlockSpec((1, NFEAT, NHID), lambda b, h: (h, 0, 0)),
            pl.BlockSpec((1, NHID, NFEAT), lambda b, h: (h, 0, 0)),
            pl.BlockSpec((1, NHID, 1), lambda b, h: (h, 0, 0)),
            pl.BlockSpec((1, 1, NHID), lambda b, h: (h, 0, 0)),
            pl.BlockSpec((1, NHID, NCLASS), lambda b, h: (h, 0, 0)),
        ],
        out_specs=pl.BlockSpec((1, 1, NCLASS), lambda b, h: (b, 0, 0)),
        out_shape=jax.ShapeDtypeStruct((B, 1, NCLASS), jnp.float32),
        compiler_params=pltpu.CompilerParams(
            dimension_semantics=("parallel", "arbitrary"),
        ),
    )(x, xt, W, wt, a1, a2, fc3d)
    return out.reshape(B, NCLASS)


# speedup vs baseline: 1.3016x; 1.0332x over previous
import jax
import jax.numpy as jnp
from jax.experimental import pallas as pl
from jax.experimental.pallas import tpu as pltpu

NFEAT = 128
NHID = 32
NCLASS = 16
NHEADS = 8
ALPHA = 0.2
B = 4
N = 1024


def _gat_kernel(x_ref, xt_ref, w_ref, wt_ref, a1_ref, a2_ref, fc_ref, out_ref):
    h_idx = pl.program_id(1)

    x = x_ref[0]
    xt = xt_ref[0]
    w = w_ref[0]
    wt = wt_ref[0]
    hm = jnp.dot(x, w, preferred_element_type=jnp.float32)
    hmt = jnp.dot(wt, xt, preferred_element_type=jnp.float32)

    f1 = jnp.dot(hm, a1_ref[0], preferred_element_type=jnp.float32)
    f2r = jnp.dot(a2_ref[0], hmt, preferred_element_type=jnp.float32)

    mx = jnp.max(f2r)
    m = f1 + mx
    m = jnp.where(m > 0, m, ALPHA * m)
    c = 1.4426950408889634
    g1 = c * (f1 - m)
    g1a = c * (ALPHA * f1 - m)
    g2 = c * f2r
    g2a = (c * ALPHA) * f2r
    p = jnp.exp2(jnp.maximum(g1 + g2, g1a + g2a))
    hm_ext = jnp.concatenate([hm, jnp.ones((N, 1), jnp.float32)], axis=1)
    num = jnp.dot(p, hm_ext, preferred_element_type=jnp.float32)
    hp = num[:, :NHID] / num[:, NHID:]
    hp = jnp.where(hp > 0, hp, jnp.exp(jnp.minimum(hp, 0.0)) - 1.0)
    pooled = jnp.max(hp, axis=0, keepdims=True)

    contrib = jnp.dot(pooled, fc_ref[0], preferred_element_type=jnp.float32)

    @pl.when(h_idx == 0)
    def _():
        out_ref[0] = contrib

    @pl.when(h_idx != 0)
    def _():
        out_ref[0] += contrib


def kernel(x, W, a, FC):
    a1 = a[:, :NHID, :]
    a2 = jnp.swapaxes(a[:, NHID:, :], 1, 2)
    wt = jnp.swapaxes(W, 1, 2)
    xt = jnp.swapaxes(x, 1, 2)
    fc3d = FC.reshape(NHEADS, NHID, NCLASS)

    out = pl.pallas_call(
        _gat_kernel,
        grid=(B, NHEADS),
        in_specs=[
            pl.BlockSpec((1, N, NFEAT), lambda b, h: (b, 0, 0)),
            pl.BlockSpec((1, NFEAT, N), lambda b, h: (b, 0, 0)),
            pl.BlockSpec((1, NFEAT, NHID), lambda b, h: (h, 0, 0)),
            pl.BlockSpec((1, NHID, NFEAT), lambda b, h: (h, 0, 0)),
            pl.BlockSpec((1, NHID, 1), lambda b, h: (h, 0, 0)),
            pl.BlockSpec((1, 1, NHID), lambda b, h: (h, 0, 0)),
            pl.BlockSpec((1, NHID, NCLASS), lambda b, h: (h, 0, 0)),
        ],
        out_specs=pl.BlockSpec((1, 1, NCLASS), lambda b, h: (b, 0, 0)),
        out_shape=jax.ShapeDtypeStruct((B, 1, NCLASS), jnp.float32),
        compiler_params=pltpu.CompilerParams(
            dimension_semantics=("parallel", "arbitrary"),
        ),
    )(x, xt, W, wt, a1, a2, fc3d)
    return out.reshape(B, NCLASS)


# speedup vs baseline: 1.5560x; 1.1954x over previous
import jax
import jax.numpy as jnp
from jax.experimental import pallas as pl
from jax.experimental.pallas import tpu as pltpu

NFEAT = 128
NHID = 32
NCLASS = 16
NHEADS = 8
ALPHA = 0.2
B = 4
N = 1024


HPG = 2


def _one_head(x, xt, w, wt, a1, a2, fc):
    hm = jnp.dot(x, w, preferred_element_type=jnp.float32)
    hmt = jnp.dot(wt, xt, preferred_element_type=jnp.float32)

    f1 = jnp.dot(hm, a1, preferred_element_type=jnp.float32)
    f2r = jnp.dot(a2, hmt, preferred_element_type=jnp.float32)

    mx = jnp.max(f2r)
    m = f1 + mx
    m = jnp.where(m > 0, m, ALPHA * m)
    c = 1.4426950408889634
    g1 = c * (f1 - m)
    g1a = c * (ALPHA * f1 - m)
    g2 = c * f2r
    g2a = (c * ALPHA) * f2r
    p = jnp.exp2(jnp.maximum(g1 + g2, g1a + g2a))
    hm_ext = jnp.concatenate([hm, jnp.ones((N, 1), jnp.float32)], axis=1)
    num = jnp.dot(p, hm_ext, preferred_element_type=jnp.float32)
    hp = num[:, :NHID] / num[:, NHID:]
    hp = jnp.where(hp > 0, hp, jnp.exp(jnp.minimum(hp, 0.0)) - 1.0)
    pooled = jnp.max(hp, axis=0, keepdims=True)
    return jnp.dot(pooled, fc, preferred_element_type=jnp.float32)


def _gat_kernel(x_ref, xt_ref, w_ref, wt_ref, a1_ref, a2_ref, fc_ref, out_ref):
    h_idx = pl.program_id(1)

    x = x_ref[0]
    xt = xt_ref[0]
    contrib = _one_head(x, xt, w_ref[0], wt_ref[0], a1_ref[0], a2_ref[0],
                        fc_ref[0])
    for k in range(1, HPG):
        contrib = contrib + _one_head(x, xt, w_ref[k], wt_ref[k], a1_ref[k],
                                      a2_ref[k], fc_ref[k])

    @pl.when(h_idx == 0)
    def _():
        out_ref[0] = contrib

    @pl.when(h_idx != 0)
    def _():
        out_ref[0] += contrib


def kernel(x, W, a, FC):
    a1 = a[:, :NHID, :]
    a2 = jnp.swapaxes(a[:, NHID:, :], 1, 2)
    wt = jnp.swapaxes(W, 1, 2)
    xt = jnp.swapaxes(x, 1, 2)
    fc3d = FC.reshape(NHEADS, NHID, NCLASS)

    out = pl.pallas_call(
        _gat_kernel,
        grid=(B, NHEADS // HPG),
        in_specs=[
            pl.BlockSpec((1, N, NFEAT), lambda b, h: (b, 0, 0)),
            pl.BlockSpec((1, NFEAT, N), lambda b, h: (b, 0, 0)),
            pl.BlockSpec((HPG, NFEAT, NHID), lambda b, h: (h, 0, 0)),
            pl.BlockSpec((HPG, NHID, NFEAT), lambda b, h: (h, 0, 0)),
            pl.BlockSpec((HPG, NHID, 1), lambda b, h: (h, 0, 0)),
            pl.BlockSpec((HPG, 1, NHID), lambda b, h: (h, 0, 0)),
            pl.BlockSpec((HPG, NHID, NCLASS), lambda b, h: (h, 0, 0)),
        ],
        out_specs=pl.BlockSpec((1, 1, NCLASS), lambda b, h: (b, 0, 0)),
        out_shape=jax.ShapeDtypeStruct((B, 1, NCLASS), jnp.float32),
        compiler_params=pltpu.CompilerParams(
            dimension_semantics=("parallel", "arbitrary"),
        ),
    )(x, xt, W, wt, a1, a2, fc3d)
    return out.reshape(B, NCLASS)
